# baseline (speedup 1.0000x reference)
import functools

import jax
import jax.numpy as jnp
from jax import lax
from jax.experimental import pallas as pl
from jax.experimental.pallas import tpu as pltpu
from jax.experimental.pallas import tpu_sc as plsc

NC = 2
NS = 16
NW = NC * NS

BBLK = 128
NBUF = 4
LAG = 3


@functools.partial(jax.jit, static_argnames=("bsz", "seq", "embed"))
def _sc_gather_t(idx3, table, *, bsz, seq, embed):
    nbt = bsz // BBLK
    n_items = seq * nbt
    items_per_w = n_items // NW
    n_groups = items_per_w // NBUF
    et = embed // 8

    mesh = plsc.VectorSubcoreMesh(
        core_axis_name="c", subcore_axis_name="s",
        num_cores=NC, num_subcores=NS)

    @functools.partial(
        pl.kernel,
        out_type=jax.ShapeDtypeStruct((seq, et, nbt, 8, BBLK), jnp.float32),
        mesh=mesh,
        scratch_types=[
            pltpu.VMEM((NBUF, BBLK), jnp.int32),
            pltpu.VMEM((NBUF, BBLK, embed), jnp.float32),
            pltpu.VMEM((NBUF, et, 8, BBLK), jnp.float32),
            pltpu.SemaphoreType.DMA((NBUF,)),
            pltpu.SemaphoreType.DMA((NBUF,)),
        ],
        compiler_params=pltpu.CompilerParams(
            use_tc_tiling_on_sc=False, needs_layout_passes=False),
    )
    def body(idx_hbm, table_hbm, out_hbm, idx_v, rows_v, tb_v, gsems, wsems):
        wid = lax.axis_index("s") * NC + lax.axis_index("c")
        k0 = wid * items_per_w

        def fetch(i, b):
            k = k0 + i
            t, bt = k // nbt, k % nbt
            pltpu.sync_copy(idx_hbm.at[t, bt], idx_v.at[b])
            pltpu.make_async_copy(
                table_hbm.at[idx_v.at[b]], rows_v.at[b], gsems.at[b]).start()

        def gather_done(i, b):
            return pltpu.make_async_copy(
                table_hbm.at[idx_v.at[b]], rows_v.at[b], gsems.at[b])

        def write(i, b):
            k = k0 + i
            t, bt = k // nbt, k % nbt
            return pltpu.make_async_copy(
                tb_v.at[b], out_hbm.at[t, :, bt], wsems.at[b])

        rg = [lax.iota(jnp.int32, 16) + g * 16 for g in range(BBLK // 16)]

        def transpose(b):
            rows = rows_v.at[b]
            tb = tb_v.at[b]

            def erow(e, c):
                col = jnp.zeros((16,), jnp.int32) + e
                for g in range(BBLK // 16):
                    vals = plsc.load_gather(rows, [rg[g], col])
                    tb[e // 8, e % 8, pl.ds(g * 16, 16)] = vals
                return c

            lax.fori_loop(0, embed, erow, 0)

        for d in range(LAG):
            fetch(d, d)

        def group(g, carry):
            for s in range(NBUF):
                i = g * NBUF + s
                @pl.when(i + LAG < items_per_w)
                def _():
                    fetch(i + LAG, (s + LAG) % NBUF)
                gather_done(i, s).wait()
                @pl.when(i >= NBUF)
                def _():
                    write(i - NBUF, s).wait()
                transpose(s)
                write(i, s).start()
            return carry

        lax.fori_loop(0, n_groups, group, 0)
        for d in range(NBUF):
            write(items_per_w - NBUF + d, d).wait()

    return body(idx3, table)


def kernel(inputs, table):
    bsz, seq = inputs.shape
    vocab, embed = table.shape
    idx3 = inputs.T.reshape(seq, bsz // BBLK, BBLK)
    out5 = _sc_gather_t(idx3, table, bsz=bsz, seq=seq, embed=embed)
    return out5.transpose(2, 4, 0, 1, 3).reshape(bsz, seq, embed)


# speedup vs baseline: 2.0663x; 2.0663x over previous
import functools

import jax
import jax.numpy as jnp
from jax import lax
from jax.experimental import pallas as pl
from jax.experimental.pallas import tpu as pltpu
from jax.experimental.pallas import tpu_sc as plsc

NC = 2
NS = 16
NW = NC * NS

CHUNK = 256
NBUF = 4
LAG = 2


@functools.partial(jax.jit, static_argnames=("n_idx", "embed"))
def _sc_gather(idx_flat, table, *, n_idx, embed):
    n_per_w = n_idx // NW
    n_chunks = n_per_w // CHUNK
    n_groups = n_chunks // NBUF
    idx_3d = idx_flat.reshape(NW, n_chunks, CHUNK)

    mesh = plsc.VectorSubcoreMesh(
        core_axis_name="c", subcore_axis_name="s",
        num_cores=NC, num_subcores=NS)

    @functools.partial(
        pl.kernel,
        out_type=jax.ShapeDtypeStruct((n_idx, 2 * embed), jnp.float32),
        mesh=mesh,
        scratch_types=[
            pltpu.VMEM((n_chunks, CHUNK), jnp.int32),
            pltpu.VMEM((NBUF, CHUNK, embed), jnp.float32),
            pltpu.SemaphoreType.DMA((NBUF,)),
            pltpu.SemaphoreType.DMA((NBUF,)),
        ],
        compiler_params=pltpu.CompilerParams(use_tc_tiling_on_sc=False),
    )
    def body(idx_hbm, table_hbm, out_hbm, idx_v, rows_v, gsems, wsems):
        wid = lax.axis_index("s") * NC + lax.axis_index("c")
        base = wid * n_per_w
        pltpu.sync_copy(idx_hbm.at[wid], idx_v)

        def gather(j, b):
            return pltpu.make_async_copy(
                table_hbm.at[idx_v.at[j]], rows_v.at[b], gsems.at[b])

        def write(j, b):
            return pltpu.make_async_copy(
                rows_v.at[b],
                out_hbm.at[pl.ds(base + j * CHUNK, CHUNK), pl.ds(0, embed)],
                wsems.at[b])

        for b in range(LAG):
            gather(b, b).start()

        def group(g, carry):
            for b in range(NBUF):
                j = g * NBUF + b
                bg = (b + LAG) % NBUF
                jg = j + LAG
                @pl.when(jg >= NBUF)
                def _():
                    write(jg - NBUF, bg).wait()
                @pl.when(jg < n_chunks)
                def _():
                    gather(jg, bg).start()
                gather(j, b).wait()
                write(j, b).start()
            return carry

        lax.fori_loop(0, n_groups, group, 0)
        for i in range(NBUF - LAG):
            j = n_chunks - (NBUF - LAG) + i
            write(j, j % NBUF).wait()

    return body(idx_3d, table)


def kernel(inputs, table):
    bsz, seq = inputs.shape
    vocab, embed = table.shape
    n_idx = bsz * seq
    out = _sc_gather(inputs.reshape(n_idx), table, n_idx=n_idx, embed=embed)
    return out[:, :embed].reshape(bsz, seq, embed)
